# baseline (speedup 1.0000x reference)
import jax
import jax.numpy as jnp
import numpy as np
from jax.experimental import pallas as pl
from jax.sharding import Mesh, PartitionSpec as P


_N = 2048


def _nnd_batch_kernel(p1_ref, p2_ref, d1_ref, d2_ref):
    p1 = p1_ref[0]
    p2 = p2_ref[0]
    n1 = jnp.sum(p1 * p1, axis=1, keepdims=True)
    n2 = jnp.sum(p2 * p2, axis=1, keepdims=True)
    ones = jnp.ones_like(n1)
    a = jnp.concatenate([p1, n1, ones], axis=1)
    b = jnp.concatenate([-2.0 * p2, ones, n2], axis=1)
    d = jax.lax.dot_general(
        a, b, (((1,), (1,)), ((), ())),
        preferred_element_type=jnp.float32,
        precision=jax.lax.Precision.HIGHEST,
    )
    d1_ref[0, 0] = jnp.min(d, axis=1)
    d2_ref[0, 0] = jnp.min(d, axis=0)


def _nnd_pallas(input1, input2):
    bsz, n, _ = input1.shape
    grid = (bsz,)
    out_shape = (
        jax.ShapeDtypeStruct((bsz, 1, n), jnp.float32),
        jax.ShapeDtypeStruct((bsz, 1, n), jnp.float32),
    )
    d1, d2 = pl.pallas_call(
        _nnd_batch_kernel,
        grid=grid,
        in_specs=[
            pl.BlockSpec((1, n, 3), lambda b: (b, 0, 0)),
            pl.BlockSpec((1, n, 3), lambda b: (b, 0, 0)),
        ],
        out_specs=(
            pl.BlockSpec((1, 1, n), lambda b: (b, 0, 0)),
            pl.BlockSpec((1, 1, n), lambda b: (b, 0, 0)),
        ),
        out_shape=out_shape,
    )(input1, input2)
    return d1.reshape(bsz, n), d2.reshape(bsz, n)


def kernel(input1, input2):
    bsz = input1.shape[0]
    ndev = min(len(jax.devices()), bsz)
    while bsz % ndev:
        ndev -= 1
    if ndev == 1:
        return _nnd_pallas(input1, input2)
    mesh = Mesh(np.array(jax.devices()[:ndev]), ("d",))
    fn = jax.shard_map(
        _nnd_pallas,
        mesh=mesh,
        in_specs=(P("d", None, None), P("d", None, None)),
        out_specs=(P("d", None), P("d", None)),
        check_vma=False,
    )
    return fn(input1, input2)


# speedup vs baseline: 7.7957x; 7.7957x over previous
import jax
import jax.numpy as jnp
import numpy as np
from jax.experimental import pallas as pl
from jax.sharding import Mesh, PartitionSpec as P


_N = 2048


def _hi_lo(x):
    hi = x.astype(jnp.bfloat16)
    lo = (x - hi.astype(jnp.float32)).astype(jnp.bfloat16)
    return hi, lo


def _nnd_batch_kernel(p1_ref, p2_ref, d1_ref, d2_ref):
    p1 = p1_ref[0]
    p2 = p2_ref[0]
    n1 = jnp.sum(p1 * p1, axis=1, keepdims=True)
    n2 = jnp.sum(p2 * p2, axis=1, keepdims=True)
    b2 = -2.0 * p2
    p1h, p1l = _hi_lo(p1)
    b2h, b2l = _hi_lo(b2)
    n1h, n1l = _hi_lo(n1)
    n2h, n2l = _hi_lo(n2)
    one = jnp.ones_like(n1h)
    a = jnp.concatenate([p1h, p1h, p1l, n1h, n1l, one, one], axis=1)
    b = jnp.concatenate([b2h, b2l, b2h, one, one, n2h, n2l], axis=1)
    d = jax.lax.dot_general(
        a, b, (((1,), (1,)), ((), ())),
        preferred_element_type=jnp.float32,
    )
    d1_ref[0, 0] = jnp.min(d, axis=1)
    d2_ref[0, 0] = jnp.min(d, axis=0)


def _nnd_pallas(input1, input2):
    bsz, n, _ = input1.shape
    grid = (bsz,)
    out_shape = (
        jax.ShapeDtypeStruct((bsz, 1, n), jnp.float32),
        jax.ShapeDtypeStruct((bsz, 1, n), jnp.float32),
    )
    d1, d2 = pl.pallas_call(
        _nnd_batch_kernel,
        grid=grid,
        in_specs=[
            pl.BlockSpec((1, n, 3), lambda b: (b, 0, 0)),
            pl.BlockSpec((1, n, 3), lambda b: (b, 0, 0)),
        ],
        out_specs=(
            pl.BlockSpec((1, 1, n), lambda b: (b, 0, 0)),
            pl.BlockSpec((1, 1, n), lambda b: (b, 0, 0)),
        ),
        out_shape=out_shape,
    )(input1, input2)
    return d1.reshape(bsz, n), d2.reshape(bsz, n)


def kernel(input1, input2):
    return _nnd_pallas(input1, input2)
